# Initial kernel scaffold; baseline (speedup 1.0000x reference)
import jax
import jax.numpy as jnp
from jax.experimental import pallas as pl


def kernel(x, batch, Wg0, bg0, Wg1, bg1, pW1, pb1, pg1, pB1, pW2, pb2, pg2, pB2, mW1, mb1, mg1, mB1, mW2, mb2, mg2, mB2, mW3, mb3):
    raise NotImplementedError("write your pallas kernel here")


# speedup vs baseline: 4.0930x; 4.0930x over previous
import functools
import jax
import jax.numpy as jnp
from jax import lax
from jax.experimental import pallas as pl
from jax.experimental.pallas import tpu as pltpu

try:
    from jax.experimental.pallas import tpu_sc as plsc
    _HAS_SC = True
except ImportError:
    _HAS_SC = False

N = 10000
IN = 128
HID = 256
OUT = 10
K = 16
NUM_GRAPHS = 64

NP = 10240
R = 256
C = 128
NRB = NP // R
RB2 = 512
NRB2 = NP // RB2
NORM = 1.0 / 17.0

BIGI = 2 ** 30


PAD = 128
W = PAD + C
NCH = W // 128
CB = 512


def _knn_body(xr_ref, xf_ref, br_ref, bc_ref, out_ref,
              vbuf_ref, ibuf_ref, nv_ref, ni_ref):
    br = br_ref[0, 0, :]
    b_lo = jnp.min(br)
    b_hi = jnp.max(br)

    def bounds(c, carry):
        lo, hi = carry
        bcv = bc_ref[:, pl.ds(c * CB, CB)]
        cio = c * CB + lax.broadcasted_iota(jnp.int32, (1, CB), 1)
        lo = jnp.minimum(lo, jnp.min(jnp.where(bcv == b_lo, cio, NP)))
        hi = jnp.maximum(hi, jnp.max(jnp.where(bcv == b_hi, cio, -1)))
        return lo, hi

    lo_col, hi_col = lax.fori_loop(0, NP // CB, bounds,
                                   (jnp.int32(NP), jnp.int32(-1)))
    t_lo = lo_col // C
    t_hi = (hi_col + C) // C

    vbuf_ref[:, :PAD] = jnp.full((R, PAD), jnp.inf, jnp.float32)
    ibuf_ref[:, :PAD] = jnp.zeros((R, PAD), jnp.int32)

    def tile_body(t, carry):
        c0 = t * C
        xc = xf_ref[pl.ds(c0, C), :]
        bcc = bc_ref[0, pl.ds(c0, C)]
        dot = lax.dot_general(xr_ref[...], xc, (((1,), (1,)), ((), ())),
                              preferred_element_type=jnp.float32)
        sqc = jnp.sum(xc * xc, axis=1)
        s = sqc[None, :] - 2.0 * dot
        same = br[:, None] == bcc[None, :]
        vbuf_ref[:, PAD:] = jnp.where(same, s, jnp.inf)
        ibuf_ref[:, PAD:] = c0 + lax.broadcasted_iota(jnp.int32, (R, C), 1)
        for k in range(K):
            def p1(c, m):
                v = vbuf_ref[:, pl.ds(c * 128, 128)]
                return jnp.minimum(m, jnp.min(v, axis=1, keepdims=True))

            m = lax.fori_loop(0, NCH, p1,
                              jnp.full((R, 1), jnp.inf, jnp.float32))

            def p2(c, mi):
                v = vbuf_ref[:, pl.ds(c * 128, 128)]
                ii = ibuf_ref[:, pl.ds(c * 128, 128)]
                cand = jnp.min(jnp.where(v == m, ii, BIGI), axis=1,
                               keepdims=True)
                return jnp.minimum(mi, cand)

            mi = lax.fori_loop(0, NCH, p2,
                               jnp.full((R, 1), BIGI, jnp.int32))

            def p3(c, z):
                v = vbuf_ref[:, pl.ds(c * 128, 128)]
                ii = ibuf_ref[:, pl.ds(c * 128, 128)]
                vbuf_ref[:, pl.ds(c * 128, 128)] = \
                    jnp.where(ii == mi, jnp.inf, v)
                return z

            lax.fori_loop(0, NCH, p3, 0)
            nv_ref[:, k:k + 1] = m
            ni_ref[:, k:k + 1] = mi
        vbuf_ref[:, :K] = nv_ref[...]
        ibuf_ref[:, :K] = ni_ref[...]
        return carry

    lax.fori_loop(t_lo, t_hi, tile_body, 0)
    out_ref[...] = ibuf_ref[:, :K]


def _knn(xp, batch3, batch_c):
    return pl.pallas_call(
        _knn_body,
        grid=(NRB,),
        in_specs=[
            pl.BlockSpec((R, IN), lambda i: (i, 0)),
            pl.BlockSpec((NP, IN), lambda i: (0, 0)),
            pl.BlockSpec((1, 1, R), lambda i: (i, 0, 0)),
            pl.BlockSpec((1, NP), lambda i: (0, 0)),
        ],
        out_specs=pl.BlockSpec((R, K), lambda i: (i, 0)),
        out_shape=jax.ShapeDtypeStruct((NP, K), jnp.int32),
        scratch_shapes=[
            pltpu.VMEM((R, W), jnp.float32),
            pltpu.VMEM((R, W), jnp.int32),
            pltpu.VMEM((R, K), jnp.float32),
            pltpu.VMEM((R, K), jnp.int32),
        ],
        compiler_params=pltpu.CompilerParams(
            dimension_semantics=("arbitrary",)),
    )(xp, xp, batch3, batch_c)


def _mm_body(x_ref, w_ref, o_ref):
    o_ref[...] = jnp.dot(x_ref[...], w_ref[...],
                         preferred_element_type=jnp.float32)


def _mm(xp, W):
    return pl.pallas_call(
        _mm_body,
        grid=(NRB2,),
        in_specs=[
            pl.BlockSpec((RB2, IN), lambda i: (i, 0)),
            pl.BlockSpec((IN, HID), lambda i: (0, 0)),
        ],
        out_specs=pl.BlockSpec((RB2, HID), lambda i: (i, 0)),
        out_shape=jax.ShapeDtypeStruct((NP, HID), jnp.float32),
    )(xp, W)


NW = 32
NPW = NP // NW
G = 8
NCHUNK = NPW // G


def _sc_gather_body(h_hbm, idx_hbm, out_hbm, idx_v, rows_v, acc_v, sem):
    c = lax.axis_index("c")
    s = lax.axis_index("s")
    wid = s * 2 + c
    base = wid * NPW

    def chunk(t, carry):
        n0 = base + t * G
        pltpu.sync_copy(idx_hbm.at[pl.ds(n0 * K, G * K)], idx_v)
        pltpu.async_copy(h_hbm.at[idx_v], rows_v, sem).wait()

        def col(ci, carry2):
            for g in range(G):
                acc = rows_v[g * K, pl.ds(ci * 16, 16)]
                for r in range(1, K):
                    acc = acc + rows_v[g * K + r, pl.ds(ci * 16, 16)]
                acc_v[g, pl.ds(ci * 16, 16)] = acc
            return carry2

        lax.fori_loop(0, HID // 16, col, 0)
        pltpu.sync_copy(acc_v, out_hbm.at[pl.ds(n0, G)])
        return carry

    lax.fori_loop(0, NCHUNK, chunk, 0)


def _sc_gather(h, idx_flat):
    mesh = plsc.VectorSubcoreMesh(core_axis_name="c", subcore_axis_name="s")
    fn = functools.partial(
        pl.kernel,
        _sc_gather_body,
        mesh=mesh,
        out_type=jax.ShapeDtypeStruct((NP, HID), jnp.float32),
        scratch_types=[
            pltpu.VMEM((G * K,), jnp.int32),
            pltpu.VMEM((G * K, HID), jnp.float32),
            pltpu.VMEM((G, HID), jnp.float32),
            pltpu.SemaphoreType.DMA,
        ],
    )()
    return fn(h, idx_flat)


def _gcn_mm_body(msg_ref, hpre_ref, b_ref, w_ref, h_out_ref, hp2_ref):
    z = (msg_ref[...] + hpre_ref[...]) * NORM + b_ref[...]
    h = jnp.where(z > 0, z, jnp.exp(jnp.where(z > 0, 0.0, z)) - 1.0)
    h_out_ref[...] = h
    hp2_ref[...] = jnp.dot(h, w_ref[...], preferred_element_type=jnp.float32)


def _gcn_mm(msg, hpre, b2d, W):
    return pl.pallas_call(
        _gcn_mm_body,
        grid=(NRB2,),
        in_specs=[
            pl.BlockSpec((RB2, HID), lambda i: (i, 0)),
            pl.BlockSpec((RB2, HID), lambda i: (i, 0)),
            pl.BlockSpec((1, HID), lambda i: (0, 0)),
            pl.BlockSpec((HID, HID), lambda i: (0, 0)),
        ],
        out_specs=[
            pl.BlockSpec((RB2, HID), lambda i: (i, 0)),
            pl.BlockSpec((RB2, HID), lambda i: (i, 0)),
        ],
        out_shape=[
            jax.ShapeDtypeStruct((NP, HID), jnp.float32),
            jax.ShapeDtypeStruct((NP, HID), jnp.float32),
        ],
    )(msg, hpre, b2d, W)


def _f1_body(msg_ref, hpre_ref, b_ref, h1_ref, w_ref, pb_ref,
             t1_ref, s_ref, ss_ref):
    i = pl.program_id(0)
    z = (msg_ref[...] + hpre_ref[...]) * NORM + b_ref[...]
    h2 = jnp.where(z > 0, z, jnp.exp(jnp.where(z > 0, 0.0, z)) - 1.0)
    hcat = jnp.concatenate([h1_ref[...], h2], axis=1)
    t1 = jnp.maximum(jnp.dot(hcat, w_ref[...],
                             preferred_element_type=jnp.float32)
                     + pb_ref[...], 0.0)
    rowid = i * RB2 + lax.broadcasted_iota(jnp.int32, (RB2, 1), 0)
    t1 = jnp.where(rowid < N, t1, 0.0)
    t1_ref[...] = t1

    @pl.when(i == 0)
    def _():
        s_ref[...] = jnp.zeros_like(s_ref)
        ss_ref[...] = jnp.zeros_like(ss_ref)

    s_ref[...] += jnp.sum(t1, axis=0, keepdims=True)
    ss_ref[...] += jnp.sum(t1 * t1, axis=0, keepdims=True)


def _f1(msg2, h2pre, b2d, h1, pW1, pb1):
    return pl.pallas_call(
        _f1_body,
        grid=(NRB2,),
        in_specs=[
            pl.BlockSpec((RB2, HID), lambda i: (i, 0)),
            pl.BlockSpec((RB2, HID), lambda i: (i, 0)),
            pl.BlockSpec((1, HID), lambda i: (0, 0)),
            pl.BlockSpec((RB2, HID), lambda i: (i, 0)),
            pl.BlockSpec((2 * HID, HID), lambda i: (0, 0)),
            pl.BlockSpec((1, HID), lambda i: (0, 0)),
        ],
        out_specs=[
            pl.BlockSpec((RB2, HID), lambda i: (i, 0)),
            pl.BlockSpec((1, HID), lambda i: (0, 0)),
            pl.BlockSpec((1, HID), lambda i: (0, 0)),
        ],
        out_shape=[
            jax.ShapeDtypeStruct((NP, HID), jnp.float32),
            jax.ShapeDtypeStruct((1, HID), jnp.float32),
            jax.ShapeDtypeStruct((1, HID), jnp.float32),
        ],
        compiler_params=pltpu.CompilerParams(
            dimension_semantics=("arbitrary",)),
    )(msg2, h2pre, b2d, h1, pW1, pb1)


def _f2_body(t1_ref, s1_ref, ss1_ref, g_ref, B_ref, w_ref, pb_ref, bt_ref,
             seg_ref, cnt_ref, s2_ref, ss2_ref):
    i = pl.program_id(0)
    mu1 = s1_ref[...] * (1.0 / N)
    var1 = ss1_ref[...] * (1.0 / N) - mu1 * mu1
    r1 = lax.rsqrt(var1 + 1e-5)
    t1n = (t1_ref[...] - mu1) * (r1 * g_ref[...]) + B_ref[...]
    t2 = jnp.maximum(jnp.dot(t1n, w_ref[...],
                             preferred_element_type=jnp.float32)
                     + pb_ref[...], 0.0)
    rowid = i * RB2 + lax.broadcasted_iota(jnp.int32, (RB2, 1), 0)
    t2 = jnp.where(rowid < N, t2, 0.0)
    br = bt_ref[0, 0, :]
    giota = lax.broadcasted_iota(jnp.int32, (RB2, NUM_GRAPHS), 1)
    oh = (br[:, None] == giota).astype(jnp.float32)

    @pl.when(i == 0)
    def _():
        seg_ref[...] = jnp.zeros_like(seg_ref)
        cnt_ref[...] = jnp.zeros_like(cnt_ref)
        s2_ref[...] = jnp.zeros_like(s2_ref)
        ss2_ref[...] = jnp.zeros_like(ss2_ref)

    seg_ref[...] += lax.dot_general(oh, t2, (((0,), (0,)), ((), ())),
                                    preferred_element_type=jnp.float32)
    cnt_ref[...] += lax.dot_general(
        oh, jnp.ones((RB2, 128), jnp.float32), (((0,), (0,)), ((), ())),
        preferred_element_type=jnp.float32)
    s2_ref[...] += jnp.sum(t2, axis=0, keepdims=True)
    ss2_ref[...] += jnp.sum(t2 * t2, axis=0, keepdims=True)


def _f2(t1, s1, ss1, pg1, pB1, pW2, pb2, batch3b):
    return pl.pallas_call(
        _f2_body,
        grid=(NRB2,),
        in_specs=[
            pl.BlockSpec((RB2, HID), lambda i: (i, 0)),
            pl.BlockSpec((1, HID), lambda i: (0, 0)),
            pl.BlockSpec((1, HID), lambda i: (0, 0)),
            pl.BlockSpec((1, HID), lambda i: (0, 0)),
            pl.BlockSpec((1, HID), lambda i: (0, 0)),
            pl.BlockSpec((HID, HID), lambda i: (0, 0)),
            pl.BlockSpec((1, HID), lambda i: (0, 0)),
            pl.BlockSpec((1, 1, RB2), lambda i: (i, 0, 0)),
        ],
        out_specs=[
            pl.BlockSpec((NUM_GRAPHS, HID), lambda i: (0, 0)),
            pl.BlockSpec((NUM_GRAPHS, 128), lambda i: (0, 0)),
            pl.BlockSpec((1, HID), lambda i: (0, 0)),
            pl.BlockSpec((1, HID), lambda i: (0, 0)),
        ],
        out_shape=[
            jax.ShapeDtypeStruct((NUM_GRAPHS, HID), jnp.float32),
            jax.ShapeDtypeStruct((NUM_GRAPHS, 128), jnp.float32),
            jax.ShapeDtypeStruct((1, HID), jnp.float32),
            jax.ShapeDtypeStruct((1, HID), jnp.float32),
        ],
        compiler_params=pltpu.CompilerParams(
            dimension_semantics=("arbitrary",)),
    )(t1, s1, ss1, pg1, pB1, pW2, pb2, batch3b)


def _head_body(seg_ref, cnt_ref, s2_ref, ss2_ref, g2_ref, B2_ref,
               w1_ref, b1_ref, g_ref, B_ref, w2_ref, b2_ref,
               gg_ref, BB_ref, w3_ref, b3_ref, o_ref):
    mu2 = s2_ref[...] * (1.0 / N)
    var2 = ss2_ref[...] * (1.0 / N) - mu2 * mu2
    r2 = lax.rsqrt(var2 + 1e-5)
    cnt = cnt_ref[:, 0:1]
    pooled = (seg_ref[...] - cnt * mu2) * (r2 * g2_ref[...]) \
        + cnt * B2_ref[...]

    def bn64(h, g, B):
        mu = jnp.mean(h, axis=0, keepdims=True)
        var = jnp.mean((h - mu) ** 2, axis=0, keepdims=True)
        return (h - mu) * lax.rsqrt(var + 1e-5) * g + B

    m = bn64(jnp.maximum(jnp.dot(pooled, w1_ref[...],
                                 preferred_element_type=jnp.float32)
                         + b1_ref[...], 0.0), g_ref[...], B_ref[...])
    m = bn64(jnp.maximum(jnp.dot(m, w2_ref[...],
                                 preferred_element_type=jnp.float32)
                         + b2_ref[...], 0.0), gg_ref[...], BB_ref[...])
    logits = jnp.dot(m, w3_ref[...],
                     preferred_element_type=jnp.float32) + b3_ref[...]
    lmax = jnp.max(logits, axis=1, keepdims=True)
    lz = logits - lmax
    o_ref[...] = lz - jnp.log(jnp.sum(jnp.exp(lz), axis=1, keepdims=True))


def _head(seg, cnt, s2, ss2, pg2, pB2, mW1, mb1, mg1, mB1,
          mW2, mb2, mg2, mB2, mW3, mb3):
    return pl.pallas_call(
        _head_body,
        out_shape=jax.ShapeDtypeStruct((NUM_GRAPHS, OUT), jnp.float32),
    )(seg, cnt, s2, ss2, pg2, pB2, mW1, mb1, mg1, mB1,
      mW2, mb2, mg2, mB2, mW3, mb3)


def _row2d(v):
    return v.reshape(1, -1)


def kernel(x, batch, Wg0, bg0, Wg1, bg1, pW1, pb1, pg1, pB1, pW2, pb2,
           pg2, pB2, mW1, mb1, mg1, mB1, mW2, mb2, mg2, mB2, mW3, mb3):
    xp = jnp.pad(x, ((0, NP - N), (0, 0)))
    bp = jnp.pad(batch.astype(jnp.int32), (0, NP - N),
                 constant_values=NUM_GRAPHS)
    batch3 = bp.reshape(NRB, 1, R)
    batch3b = bp.reshape(NRB2, 1, RB2)
    batch_c = bp.reshape(1, NP)

    idx = _knn(xp, batch3, batch_c)
    idx_flat = idx.reshape(-1)

    h1pre = _mm(xp, Wg0)
    msg1 = _sc_gather(h1pre, idx_flat)
    h1, h2pre = _gcn_mm(msg1, h1pre, _row2d(bg0), Wg1)
    msg2 = _sc_gather(h2pre, idx_flat)
    t1, s1, ss1 = _f1(msg2, h2pre, _row2d(bg1), h1, pW1, _row2d(pb1))
    seg, cnt, s2, ss2 = _f2(t1, s1, ss1, _row2d(pg1), _row2d(pB1),
                            pW2, _row2d(pb2), batch3b)
    return _head(seg, cnt, s2, ss2, _row2d(pg2), _row2d(pB2),
                 mW1, _row2d(mb1), _row2d(mg1), _row2d(mB1),
                 mW2, _row2d(mb2), _row2d(mg2), _row2d(mB2),
                 mW3, _row2d(mb3))


# speedup vs baseline: 5.4499x; 1.3315x over previous
import functools
import jax
import jax.numpy as jnp
from jax import lax
from jax.experimental import pallas as pl
from jax.experimental.pallas import tpu as pltpu

try:
    from jax.experimental.pallas import tpu_sc as plsc
    _HAS_SC = True
except ImportError:
    _HAS_SC = False

N = 10000
IN = 128
HID = 256
OUT = 10
K = 16
NUM_GRAPHS = 64

NP = 10240
R = 256
C = 128
NRB = NP // R
RB2 = 512
NRB2 = NP // RB2
NORM = 1.0 / 17.0

BIGI = 2 ** 30


CB = 512


def _knn_body(xr_ref, xf_ref, br_ref, bc_ref, out_ref, vbuf_ref, ni_ref):
    br = br_ref[0, 0, :]
    b_lo = jnp.min(br)
    b_hi = jnp.max(br)

    def bounds(c, carry):
        lo, hi = carry
        bcv = bc_ref[:, pl.ds(c * CB, CB)]
        cio = c * CB + lax.broadcasted_iota(jnp.int32, (1, CB), 1)
        lo = jnp.minimum(lo, jnp.min(jnp.where(bcv == b_lo, cio, NP)))
        hi = jnp.maximum(hi, jnp.max(jnp.where(bcv == b_hi, cio, -1)))
        return lo, hi

    lo_col, hi_col = lax.fori_loop(0, NP // CB, bounds,
                                   (jnp.int32(NP), jnp.int32(-1)))
    ch_lo = lo_col // C
    ch_hi = (hi_col + C) // C

    def fill(t, carry):
        c0 = t * C
        xc = xf_ref[pl.ds(c0, C), :]
        bcc = bc_ref[0, pl.ds(c0, C)]
        dot = lax.dot_general(xr_ref[...], xc, (((1,), (1,)), ((), ())),
                              preferred_element_type=jnp.float32)
        sqc = jnp.sum(xc * xc, axis=1)
        s = sqc[None, :] - 2.0 * dot
        same = br[:, None] == bcc[None, :]
        vbuf_ref[:, pl.ds(c0, C)] = jnp.where(same, s, jnp.inf)
        return carry

    lax.fori_loop(ch_lo, ch_hi, fill, 0)

    def sweep(prev_mi, clear):
        def chunk(t, carry):
            m, mi = carry
            c0 = t * C
            v = vbuf_ref[:, pl.ds(c0, C)]
            ii = c0 + lax.broadcasted_iota(jnp.int32, (R, C), 1)
            if clear:
                v = jnp.where(ii == prev_mi, jnp.inf, v)
                vbuf_ref[:, pl.ds(c0, C)] = v
            mc = jnp.min(v, axis=1, keepdims=True)
            mic = jnp.min(jnp.where(v == mc, ii, BIGI), axis=1,
                          keepdims=True)
            mi = jnp.where(mc < m, mic, mi)
            m = jnp.minimum(m, mc)
            return m, mi

        return lax.fori_loop(ch_lo, ch_hi, chunk,
                             (jnp.full((R, 1), jnp.inf, jnp.float32),
                              jnp.zeros((R, 1), jnp.int32)))

    _, mi = sweep(None, False)
    ni_ref[:, 0:1] = mi
    for k in range(1, K):
        _, mi = sweep(mi, True)
        ni_ref[:, k:k + 1] = mi
    out_ref[...] = ni_ref[...]


def _knn(xp, batch3, batch_c):
    return pl.pallas_call(
        _knn_body,
        grid=(NRB,),
        in_specs=[
            pl.BlockSpec((R, IN), lambda i: (i, 0)),
            pl.BlockSpec((NP, IN), lambda i: (0, 0)),
            pl.BlockSpec((1, 1, R), lambda i: (i, 0, 0)),
            pl.BlockSpec((1, NP), lambda i: (0, 0)),
        ],
        out_specs=pl.BlockSpec((R, K), lambda i: (i, 0)),
        out_shape=jax.ShapeDtypeStruct((NP, K), jnp.int32),
        scratch_shapes=[
            pltpu.VMEM((R, NP), jnp.float32),
            pltpu.VMEM((R, K), jnp.int32),
        ],
        compiler_params=pltpu.CompilerParams(
            dimension_semantics=("arbitrary",)),
    )(xp, xp, batch3, batch_c)


def _mm_body(x_ref, w_ref, o_ref):
    o_ref[...] = jnp.dot(x_ref[...], w_ref[...],
                         preferred_element_type=jnp.float32)


def _mm(xp, W):
    return pl.pallas_call(
        _mm_body,
        grid=(NRB2,),
        in_specs=[
            pl.BlockSpec((RB2, IN), lambda i: (i, 0)),
            pl.BlockSpec((IN, HID), lambda i: (0, 0)),
        ],
        out_specs=pl.BlockSpec((RB2, HID), lambda i: (i, 0)),
        out_shape=jax.ShapeDtypeStruct((NP, HID), jnp.float32),
    )(xp, W)


NW = 32
NPW = NP // NW
G = 8
NCHUNK = NPW // G


def _sc_gather_body(h_hbm, idx_hbm, out_hbm, idx_v, rows_v, acc_v, sem):
    c = lax.axis_index("c")
    s = lax.axis_index("s")
    wid = s * 2 + c
    base = wid * NPW

    def chunk(t, carry):
        n0 = base + t * G
        pltpu.sync_copy(idx_hbm.at[pl.ds(n0 * K, G * K)], idx_v)
        pltpu.async_copy(h_hbm.at[idx_v], rows_v, sem).wait()

        def col(ci, carry2):
            for g in range(G):
                acc = rows_v[g * K, pl.ds(ci * 16, 16)]
                for r in range(1, K):
                    acc = acc + rows_v[g * K + r, pl.ds(ci * 16, 16)]
                acc_v[g, pl.ds(ci * 16, 16)] = acc
            return carry2

        lax.fori_loop(0, HID // 16, col, 0)
        pltpu.sync_copy(acc_v, out_hbm.at[pl.ds(n0, G)])
        return carry

    lax.fori_loop(0, NCHUNK, chunk, 0)


def _sc_gather(h, idx_flat):
    mesh = plsc.VectorSubcoreMesh(core_axis_name="c", subcore_axis_name="s")
    fn = functools.partial(
        pl.kernel,
        _sc_gather_body,
        mesh=mesh,
        out_type=jax.ShapeDtypeStruct((NP, HID), jnp.float32),
        scratch_types=[
            pltpu.VMEM((G * K,), jnp.int32),
            pltpu.VMEM((G * K, HID), jnp.float32),
            pltpu.VMEM((G, HID), jnp.float32),
            pltpu.SemaphoreType.DMA,
        ],
    )()
    return fn(h, idx_flat)


def _gcn_mm_body(msg_ref, hpre_ref, b_ref, w_ref, h_out_ref, hp2_ref):
    z = (msg_ref[...] + hpre_ref[...]) * NORM + b_ref[...]
    h = jnp.where(z > 0, z, jnp.exp(jnp.where(z > 0, 0.0, z)) - 1.0)
    h_out_ref[...] = h
    hp2_ref[...] = jnp.dot(h, w_ref[...], preferred_element_type=jnp.float32)


def _gcn_mm(msg, hpre, b2d, W):
    return pl.pallas_call(
        _gcn_mm_body,
        grid=(NRB2,),
        in_specs=[
            pl.BlockSpec((RB2, HID), lambda i: (i, 0)),
            pl.BlockSpec((RB2, HID), lambda i: (i, 0)),
            pl.BlockSpec((1, HID), lambda i: (0, 0)),
            pl.BlockSpec((HID, HID), lambda i: (0, 0)),
        ],
        out_specs=[
            pl.BlockSpec((RB2, HID), lambda i: (i, 0)),
            pl.BlockSpec((RB2, HID), lambda i: (i, 0)),
        ],
        out_shape=[
            jax.ShapeDtypeStruct((NP, HID), jnp.float32),
            jax.ShapeDtypeStruct((NP, HID), jnp.float32),
        ],
    )(msg, hpre, b2d, W)


def _f1_body(msg_ref, hpre_ref, b_ref, h1_ref, w_ref, pb_ref,
             t1_ref, s_ref, ss_ref):
    i = pl.program_id(0)
    z = (msg_ref[...] + hpre_ref[...]) * NORM + b_ref[...]
    h2 = jnp.where(z > 0, z, jnp.exp(jnp.where(z > 0, 0.0, z)) - 1.0)
    hcat = jnp.concatenate([h1_ref[...], h2], axis=1)
    t1 = jnp.maximum(jnp.dot(hcat, w_ref[...],
                             preferred_element_type=jnp.float32)
                     + pb_ref[...], 0.0)
    rowid = i * RB2 + lax.broadcasted_iota(jnp.int32, (RB2, 1), 0)
    t1 = jnp.where(rowid < N, t1, 0.0)
    t1_ref[...] = t1

    @pl.when(i == 0)
    def _():
        s_ref[...] = jnp.zeros_like(s_ref)
        ss_ref[...] = jnp.zeros_like(ss_ref)

    s_ref[...] += jnp.sum(t1, axis=0, keepdims=True)
    ss_ref[...] += jnp.sum(t1 * t1, axis=0, keepdims=True)


def _f1(msg2, h2pre, b2d, h1, pW1, pb1):
    return pl.pallas_call(
        _f1_body,
        grid=(NRB2,),
        in_specs=[
            pl.BlockSpec((RB2, HID), lambda i: (i, 0)),
            pl.BlockSpec((RB2, HID), lambda i: (i, 0)),
            pl.BlockSpec((1, HID), lambda i: (0, 0)),
            pl.BlockSpec((RB2, HID), lambda i: (i, 0)),
            pl.BlockSpec((2 * HID, HID), lambda i: (0, 0)),
            pl.BlockSpec((1, HID), lambda i: (0, 0)),
        ],
        out_specs=[
            pl.BlockSpec((RB2, HID), lambda i: (i, 0)),
            pl.BlockSpec((1, HID), lambda i: (0, 0)),
            pl.BlockSpec((1, HID), lambda i: (0, 0)),
        ],
        out_shape=[
            jax.ShapeDtypeStruct((NP, HID), jnp.float32),
            jax.ShapeDtypeStruct((1, HID), jnp.float32),
            jax.ShapeDtypeStruct((1, HID), jnp.float32),
        ],
        compiler_params=pltpu.CompilerParams(
            dimension_semantics=("arbitrary",)),
    )(msg2, h2pre, b2d, h1, pW1, pb1)


def _f2_body(t1_ref, s1_ref, ss1_ref, g_ref, B_ref, w_ref, pb_ref, bt_ref,
             seg_ref, cnt_ref, s2_ref, ss2_ref):
    i = pl.program_id(0)
    mu1 = s1_ref[...] * (1.0 / N)
    var1 = ss1_ref[...] * (1.0 / N) - mu1 * mu1
    r1 = lax.rsqrt(var1 + 1e-5)
    t1n = (t1_ref[...] - mu1) * (r1 * g_ref[...]) + B_ref[...]
    t2 = jnp.maximum(jnp.dot(t1n, w_ref[...],
                             preferred_element_type=jnp.float32)
                     + pb_ref[...], 0.0)
    rowid = i * RB2 + lax.broadcasted_iota(jnp.int32, (RB2, 1), 0)
    t2 = jnp.where(rowid < N, t2, 0.0)
    br = bt_ref[0, 0, :]
    giota = lax.broadcasted_iota(jnp.int32, (RB2, NUM_GRAPHS), 1)
    oh = (br[:, None] == giota).astype(jnp.float32)

    @pl.when(i == 0)
    def _():
        seg_ref[...] = jnp.zeros_like(seg_ref)
        cnt_ref[...] = jnp.zeros_like(cnt_ref)
        s2_ref[...] = jnp.zeros_like(s2_ref)
        ss2_ref[...] = jnp.zeros_like(ss2_ref)

    seg_ref[...] += lax.dot_general(oh, t2, (((0,), (0,)), ((), ())),
                                    preferred_element_type=jnp.float32)
    cnt_ref[...] += lax.dot_general(
        oh, jnp.ones((RB2, 128), jnp.float32), (((0,), (0,)), ((), ())),
        preferred_element_type=jnp.float32)
    s2_ref[...] += jnp.sum(t2, axis=0, keepdims=True)
    ss2_ref[...] += jnp.sum(t2 * t2, axis=0, keepdims=True)


def _f2(t1, s1, ss1, pg1, pB1, pW2, pb2, batch3b):
    return pl.pallas_call(
        _f2_body,
        grid=(NRB2,),
        in_specs=[
            pl.BlockSpec((RB2, HID), lambda i: (i, 0)),
            pl.BlockSpec((1, HID), lambda i: (0, 0)),
            pl.BlockSpec((1, HID), lambda i: (0, 0)),
            pl.BlockSpec((1, HID), lambda i: (0, 0)),
            pl.BlockSpec((1, HID), lambda i: (0, 0)),
            pl.BlockSpec((HID, HID), lambda i: (0, 0)),
            pl.BlockSpec((1, HID), lambda i: (0, 0)),
            pl.BlockSpec((1, 1, RB2), lambda i: (i, 0, 0)),
        ],
        out_specs=[
            pl.BlockSpec((NUM_GRAPHS, HID), lambda i: (0, 0)),
            pl.BlockSpec((NUM_GRAPHS, 128), lambda i: (0, 0)),
            pl.BlockSpec((1, HID), lambda i: (0, 0)),
            pl.BlockSpec((1, HID), lambda i: (0, 0)),
        ],
        out_shape=[
            jax.ShapeDtypeStruct((NUM_GRAPHS, HID), jnp.float32),
            jax.ShapeDtypeStruct((NUM_GRAPHS, 128), jnp.float32),
            jax.ShapeDtypeStruct((1, HID), jnp.float32),
            jax.ShapeDtypeStruct((1, HID), jnp.float32),
        ],
        compiler_params=pltpu.CompilerParams(
            dimension_semantics=("arbitrary",)),
    )(t1, s1, ss1, pg1, pB1, pW2, pb2, batch3b)


def _head_body(seg_ref, cnt_ref, s2_ref, ss2_ref, g2_ref, B2_ref,
               w1_ref, b1_ref, g_ref, B_ref, w2_ref, b2_ref,
               gg_ref, BB_ref, w3_ref, b3_ref, o_ref):
    mu2 = s2_ref[...] * (1.0 / N)
    var2 = ss2_ref[...] * (1.0 / N) - mu2 * mu2
    r2 = lax.rsqrt(var2 + 1e-5)
    cnt = cnt_ref[:, 0:1]
    pooled = (seg_ref[...] - cnt * mu2) * (r2 * g2_ref[...]) \
        + cnt * B2_ref[...]

    def bn64(h, g, B):
        mu = jnp.mean(h, axis=0, keepdims=True)
        var = jnp.mean((h - mu) ** 2, axis=0, keepdims=True)
        return (h - mu) * lax.rsqrt(var + 1e-5) * g + B

    m = bn64(jnp.maximum(jnp.dot(pooled, w1_ref[...],
                                 preferred_element_type=jnp.float32)
                         + b1_ref[...], 0.0), g_ref[...], B_ref[...])
    m = bn64(jnp.maximum(jnp.dot(m, w2_ref[...],
                                 preferred_element_type=jnp.float32)
                         + b2_ref[...], 0.0), gg_ref[...], BB_ref[...])
    logits = jnp.dot(m, w3_ref[...],
                     preferred_element_type=jnp.float32) + b3_ref[...]
    lmax = jnp.max(logits, axis=1, keepdims=True)
    lz = logits - lmax
    o_ref[...] = lz - jnp.log(jnp.sum(jnp.exp(lz), axis=1, keepdims=True))


def _head(seg, cnt, s2, ss2, pg2, pB2, mW1, mb1, mg1, mB1,
          mW2, mb2, mg2, mB2, mW3, mb3):
    return pl.pallas_call(
        _head_body,
        out_shape=jax.ShapeDtypeStruct((NUM_GRAPHS, OUT), jnp.float32),
    )(seg, cnt, s2, ss2, pg2, pB2, mW1, mb1, mg1, mB1,
      mW2, mb2, mg2, mB2, mW3, mb3)


def _row2d(v):
    return v.reshape(1, -1)


def kernel(x, batch, Wg0, bg0, Wg1, bg1, pW1, pb1, pg1, pB1, pW2, pb2,
           pg2, pB2, mW1, mb1, mg1, mB1, mW2, mb2, mg2, mB2, mW3, mb3):
    xp = jnp.pad(x, ((0, NP - N), (0, 0)))
    bp = jnp.pad(batch.astype(jnp.int32), (0, NP - N),
                 constant_values=NUM_GRAPHS)
    batch3 = bp.reshape(NRB, 1, R)
    batch3b = bp.reshape(NRB2, 1, RB2)
    batch_c = bp.reshape(1, NP)

    idx = _knn(xp, batch3, batch_c)
    idx_flat = idx.reshape(-1)

    h1pre = _mm(xp, Wg0)
    msg1 = _sc_gather(h1pre, idx_flat)
    h1, h2pre = _gcn_mm(msg1, h1pre, _row2d(bg0), Wg1)
    msg2 = _sc_gather(h2pre, idx_flat)
    t1, s1, ss1 = _f1(msg2, h2pre, _row2d(bg1), h1, pW1, _row2d(pb1))
    seg, cnt, s2, ss2 = _f2(t1, s1, ss1, _row2d(pg1), _row2d(pB1),
                            pW2, _row2d(pb2), batch3b)
    return _head(seg, cnt, s2, ss2, _row2d(pg2), _row2d(pB2),
                 mW1, _row2d(mb1), _row2d(mg1), _row2d(mB1),
                 mW2, _row2d(mb2), _row2d(mg2), _row2d(mB2),
                 mW3, _row2d(mb3))


# speedup vs baseline: 5.7750x; 1.0597x over previous
import functools
import jax
import jax.numpy as jnp
from jax import lax
from jax.experimental import pallas as pl
from jax.experimental.pallas import tpu as pltpu

try:
    from jax.experimental.pallas import tpu_sc as plsc
    _HAS_SC = True
except ImportError:
    _HAS_SC = False

N = 10000
IN = 128
HID = 256
OUT = 10
K = 16
NUM_GRAPHS = 64

NP = 10240
R = 256
C = 128
NRB = NP // R
RB2 = 512
NRB2 = NP // RB2
NORM = 1.0 / 17.0

BIGI = 2 ** 30


def _knn_body(xr_ref, xf_ref, br_ref, bc_ref, b2_ref, out_ref,
              vbuf_ref, ni_ref):
    br = br_ref[0, 0, :]
    b_lo = jnp.min(br)
    b_hi = jnp.max(br)
    b2 = b2_ref[...]
    lo_col = jnp.sum((b2 < b_lo).astype(jnp.int32))
    hi_col = jnp.sum((b2 <= b_hi).astype(jnp.int32))
    ch_lo = lo_col // C
    ch_hi = (hi_col + C - 1) // C

    def fill(t, carry):
        c0 = t * C
        xc = xf_ref[pl.ds(c0, C), :]
        bcc = bc_ref[0, pl.ds(c0, C)]
        dot = lax.dot_general(xr_ref[...], xc, (((1,), (1,)), ((), ())),
                              preferred_element_type=jnp.float32)
        sqc = jnp.sum(xc * xc, axis=1)
        s = sqc[None, :] - 2.0 * dot
        same = br[:, None] == bcc[None, :]
        vbuf_ref[:, pl.ds(c0, C)] = jnp.where(same, s, jnp.inf)
        return carry

    lax.fori_loop(ch_lo, ch_hi, fill, 0)

    def sweep(prev_mi, clear):
        def chunk(t, carry):
            m, mi = carry
            c0 = t * C
            v = vbuf_ref[:, pl.ds(c0, C)]
            ii = c0 + lax.broadcasted_iota(jnp.int32, (R, C), 1)
            if clear:
                v = jnp.where(ii == prev_mi, jnp.inf, v)
                vbuf_ref[:, pl.ds(c0, C)] = v
            mc = jnp.min(v, axis=1, keepdims=True)
            mic = jnp.min(jnp.where(v == mc, ii, BIGI), axis=1,
                          keepdims=True)
            mi = jnp.where(mc < m, mic, mi)
            m = jnp.minimum(m, mc)
            return m, mi

        return lax.fori_loop(ch_lo, ch_hi, chunk,
                             (jnp.full((R, 1), jnp.inf, jnp.float32),
                              jnp.zeros((R, 1), jnp.int32)))

    _, mi = sweep(None, False)
    ni_ref[:, 0:1] = mi
    for k in range(1, K):
        _, mi = sweep(mi, True)
        ni_ref[:, k:k + 1] = mi
    out_ref[...] = ni_ref[...]


def _knn(xp, batch3, batch_c, batch2):
    return pl.pallas_call(
        _knn_body,
        grid=(NRB,),
        in_specs=[
            pl.BlockSpec((R, IN), lambda i: (i, 0)),
            pl.BlockSpec((NP, IN), lambda i: (0, 0)),
            pl.BlockSpec((1, 1, R), lambda i: (i, 0, 0)),
            pl.BlockSpec((1, NP), lambda i: (0, 0)),
            pl.BlockSpec((8, NP // 8), lambda i: (0, 0)),
        ],
        out_specs=pl.BlockSpec((R, K), lambda i: (i, 0)),
        out_shape=jax.ShapeDtypeStruct((NP, K), jnp.int32),
        scratch_shapes=[
            pltpu.VMEM((R, NP), jnp.float32),
            pltpu.VMEM((R, K), jnp.int32),
        ],
        compiler_params=pltpu.CompilerParams(
            dimension_semantics=("arbitrary",)),
    )(xp, xp, batch3, batch_c, batch2)


def _mm_body(x_ref, w_ref, o_ref):
    o_ref[...] = jnp.dot(x_ref[...], w_ref[...],
                         preferred_element_type=jnp.float32)


def _mm(xp, W):
    return pl.pallas_call(
        _mm_body,
        grid=(NRB2,),
        in_specs=[
            pl.BlockSpec((RB2, IN), lambda i: (i, 0)),
            pl.BlockSpec((IN, HID), lambda i: (0, 0)),
        ],
        out_specs=pl.BlockSpec((RB2, HID), lambda i: (i, 0)),
        out_shape=jax.ShapeDtypeStruct((NP, HID), jnp.float32),
    )(xp, W)


NW = 32
NPW = NP // NW
G = 8
NCHUNK = NPW // G


def _sc_gather_body(h_hbm, idx_hbm, out_hbm, idx_v, rows_v, acc_v, sem):
    c = lax.axis_index("c")
    s = lax.axis_index("s")
    wid = s * 2 + c
    base = wid * NPW

    def chunk(t, carry):
        n0 = base + t * G
        pltpu.sync_copy(idx_hbm.at[pl.ds(n0 * K, G * K)], idx_v)
        pltpu.async_copy(h_hbm.at[idx_v], rows_v, sem).wait()

        def col(ci, carry2):
            for g in range(G):
                acc = rows_v[g * K, pl.ds(ci * 16, 16)]
                for r in range(1, K):
                    acc = acc + rows_v[g * K + r, pl.ds(ci * 16, 16)]
                acc_v[g, pl.ds(ci * 16, 16)] = acc
            return carry2

        lax.fori_loop(0, HID // 16, col, 0)
        pltpu.sync_copy(acc_v, out_hbm.at[pl.ds(n0, G)])
        return carry

    lax.fori_loop(0, NCHUNK, chunk, 0)


def _sc_gather(h, idx_flat):
    mesh = plsc.VectorSubcoreMesh(core_axis_name="c", subcore_axis_name="s")
    fn = functools.partial(
        pl.kernel,
        _sc_gather_body,
        mesh=mesh,
        out_type=jax.ShapeDtypeStruct((NP, HID), jnp.float32),
        scratch_types=[
            pltpu.VMEM((G * K,), jnp.int32),
            pltpu.VMEM((G * K, HID), jnp.float32),
            pltpu.VMEM((G, HID), jnp.float32),
            pltpu.SemaphoreType.DMA,
        ],
    )()
    return fn(h, idx_flat)


def _gcn_mm_body(msg_ref, hpre_ref, b_ref, w_ref, h_out_ref, hp2_ref):
    z = (msg_ref[...] + hpre_ref[...]) * NORM + b_ref[...]
    h = jnp.where(z > 0, z, jnp.exp(jnp.where(z > 0, 0.0, z)) - 1.0)
    h_out_ref[...] = h
    hp2_ref[...] = jnp.dot(h, w_ref[...], preferred_element_type=jnp.float32)


def _gcn_mm(msg, hpre, b2d, W):
    return pl.pallas_call(
        _gcn_mm_body,
        grid=(NRB2,),
        in_specs=[
            pl.BlockSpec((RB2, HID), lambda i: (i, 0)),
            pl.BlockSpec((RB2, HID), lambda i: (i, 0)),
            pl.BlockSpec((1, HID), lambda i: (0, 0)),
            pl.BlockSpec((HID, HID), lambda i: (0, 0)),
        ],
        out_specs=[
            pl.BlockSpec((RB2, HID), lambda i: (i, 0)),
            pl.BlockSpec((RB2, HID), lambda i: (i, 0)),
        ],
        out_shape=[
            jax.ShapeDtypeStruct((NP, HID), jnp.float32),
            jax.ShapeDtypeStruct((NP, HID), jnp.float32),
        ],
    )(msg, hpre, b2d, W)


def _f1_body(msg_ref, hpre_ref, b_ref, h1_ref, w_ref, pb_ref,
             t1_ref, s_ref, ss_ref):
    i = pl.program_id(0)
    z = (msg_ref[...] + hpre_ref[...]) * NORM + b_ref[...]
    h2 = jnp.where(z > 0, z, jnp.exp(jnp.where(z > 0, 0.0, z)) - 1.0)
    hcat = jnp.concatenate([h1_ref[...], h2], axis=1)
    t1 = jnp.maximum(jnp.dot(hcat, w_ref[...],
                             preferred_element_type=jnp.float32)
                     + pb_ref[...], 0.0)
    rowid = i * RB2 + lax.broadcasted_iota(jnp.int32, (RB2, 1), 0)
    t1 = jnp.where(rowid < N, t1, 0.0)
    t1_ref[...] = t1

    @pl.when(i == 0)
    def _():
        s_ref[...] = jnp.zeros_like(s_ref)
        ss_ref[...] = jnp.zeros_like(ss_ref)

    s_ref[...] += jnp.sum(t1, axis=0, keepdims=True)
    ss_ref[...] += jnp.sum(t1 * t1, axis=0, keepdims=True)


def _f1(msg2, h2pre, b2d, h1, pW1, pb1):
    return pl.pallas_call(
        _f1_body,
        grid=(NRB2,),
        in_specs=[
            pl.BlockSpec((RB2, HID), lambda i: (i, 0)),
            pl.BlockSpec((RB2, HID), lambda i: (i, 0)),
            pl.BlockSpec((1, HID), lambda i: (0, 0)),
            pl.BlockSpec((RB2, HID), lambda i: (i, 0)),
            pl.BlockSpec((2 * HID, HID), lambda i: (0, 0)),
            pl.BlockSpec((1, HID), lambda i: (0, 0)),
        ],
        out_specs=[
            pl.BlockSpec((RB2, HID), lambda i: (i, 0)),
            pl.BlockSpec((1, HID), lambda i: (0, 0)),
            pl.BlockSpec((1, HID), lambda i: (0, 0)),
        ],
        out_shape=[
            jax.ShapeDtypeStruct((NP, HID), jnp.float32),
            jax.ShapeDtypeStruct((1, HID), jnp.float32),
            jax.ShapeDtypeStruct((1, HID), jnp.float32),
        ],
        compiler_params=pltpu.CompilerParams(
            dimension_semantics=("arbitrary",)),
    )(msg2, h2pre, b2d, h1, pW1, pb1)


def _f2_body(t1_ref, s1_ref, ss1_ref, g_ref, B_ref, w_ref, pb_ref, bt_ref,
             seg_ref, cnt_ref, s2_ref, ss2_ref):
    i = pl.program_id(0)
    mu1 = s1_ref[...] * (1.0 / N)
    var1 = ss1_ref[...] * (1.0 / N) - mu1 * mu1
    r1 = lax.rsqrt(var1 + 1e-5)
    t1n = (t1_ref[...] - mu1) * (r1 * g_ref[...]) + B_ref[...]
    t2 = jnp.maximum(jnp.dot(t1n, w_ref[...],
                             preferred_element_type=jnp.float32)
                     + pb_ref[...], 0.0)
    rowid = i * RB2 + lax.broadcasted_iota(jnp.int32, (RB2, 1), 0)
    t2 = jnp.where(rowid < N, t2, 0.0)
    br = bt_ref[0, 0, :]
    giota = lax.broadcasted_iota(jnp.int32, (RB2, NUM_GRAPHS), 1)
    oh = (br[:, None] == giota).astype(jnp.float32)

    @pl.when(i == 0)
    def _():
        seg_ref[...] = jnp.zeros_like(seg_ref)
        cnt_ref[...] = jnp.zeros_like(cnt_ref)
        s2_ref[...] = jnp.zeros_like(s2_ref)
        ss2_ref[...] = jnp.zeros_like(ss2_ref)

    seg_ref[...] += lax.dot_general(oh, t2, (((0,), (0,)), ((), ())),
                                    preferred_element_type=jnp.float32)
    cnt_ref[...] += lax.dot_general(
        oh, jnp.ones((RB2, 128), jnp.float32), (((0,), (0,)), ((), ())),
        preferred_element_type=jnp.float32)
    s2_ref[...] += jnp.sum(t2, axis=0, keepdims=True)
    ss2_ref[...] += jnp.sum(t2 * t2, axis=0, keepdims=True)


def _f2(t1, s1, ss1, pg1, pB1, pW2, pb2, batch3b):
    return pl.pallas_call(
        _f2_body,
        grid=(NRB2,),
        in_specs=[
            pl.BlockSpec((RB2, HID), lambda i: (i, 0)),
            pl.BlockSpec((1, HID), lambda i: (0, 0)),
            pl.BlockSpec((1, HID), lambda i: (0, 0)),
            pl.BlockSpec((1, HID), lambda i: (0, 0)),
            pl.BlockSpec((1, HID), lambda i: (0, 0)),
            pl.BlockSpec((HID, HID), lambda i: (0, 0)),
            pl.BlockSpec((1, HID), lambda i: (0, 0)),
            pl.BlockSpec((1, 1, RB2), lambda i: (i, 0, 0)),
        ],
        out_specs=[
            pl.BlockSpec((NUM_GRAPHS, HID), lambda i: (0, 0)),
            pl.BlockSpec((NUM_GRAPHS, 128), lambda i: (0, 0)),
            pl.BlockSpec((1, HID), lambda i: (0, 0)),
            pl.BlockSpec((1, HID), lambda i: (0, 0)),
        ],
        out_shape=[
            jax.ShapeDtypeStruct((NUM_GRAPHS, HID), jnp.float32),
            jax.ShapeDtypeStruct((NUM_GRAPHS, 128), jnp.float32),
            jax.ShapeDtypeStruct((1, HID), jnp.float32),
            jax.ShapeDtypeStruct((1, HID), jnp.float32),
        ],
        compiler_params=pltpu.CompilerParams(
            dimension_semantics=("arbitrary",)),
    )(t1, s1, ss1, pg1, pB1, pW2, pb2, batch3b)


def _head_body(seg_ref, cnt_ref, s2_ref, ss2_ref, g2_ref, B2_ref,
               w1_ref, b1_ref, g_ref, B_ref, w2_ref, b2_ref,
               gg_ref, BB_ref, w3_ref, b3_ref, o_ref):
    mu2 = s2_ref[...] * (1.0 / N)
    var2 = ss2_ref[...] * (1.0 / N) - mu2 * mu2
    r2 = lax.rsqrt(var2 + 1e-5)
    cnt = cnt_ref[:, 0:1]
    pooled = (seg_ref[...] - cnt * mu2) * (r2 * g2_ref[...]) \
        + cnt * B2_ref[...]

    def bn64(h, g, B):
        mu = jnp.mean(h, axis=0, keepdims=True)
        var = jnp.mean((h - mu) ** 2, axis=0, keepdims=True)
        return (h - mu) * lax.rsqrt(var + 1e-5) * g + B

    m = bn64(jnp.maximum(jnp.dot(pooled, w1_ref[...],
                                 preferred_element_type=jnp.float32)
                         + b1_ref[...], 0.0), g_ref[...], B_ref[...])
    m = bn64(jnp.maximum(jnp.dot(m, w2_ref[...],
                                 preferred_element_type=jnp.float32)
                         + b2_ref[...], 0.0), gg_ref[...], BB_ref[...])
    logits = jnp.dot(m, w3_ref[...],
                     preferred_element_type=jnp.float32) + b3_ref[...]
    lmax = jnp.max(logits, axis=1, keepdims=True)
    lz = logits - lmax
    o_ref[...] = lz - jnp.log(jnp.sum(jnp.exp(lz), axis=1, keepdims=True))


def _head(seg, cnt, s2, ss2, pg2, pB2, mW1, mb1, mg1, mB1,
          mW2, mb2, mg2, mB2, mW3, mb3):
    return pl.pallas_call(
        _head_body,
        out_shape=jax.ShapeDtypeStruct((NUM_GRAPHS, OUT), jnp.float32),
    )(seg, cnt, s2, ss2, pg2, pB2, mW1, mb1, mg1, mB1,
      mW2, mb2, mg2, mB2, mW3, mb3)


def _row2d(v):
    return v.reshape(1, -1)


def kernel(x, batch, Wg0, bg0, Wg1, bg1, pW1, pb1, pg1, pB1, pW2, pb2,
           pg2, pB2, mW1, mb1, mg1, mB1, mW2, mb2, mg2, mB2, mW3, mb3):
    xp = jnp.pad(x, ((0, NP - N), (0, 0)))
    bp = jnp.pad(batch.astype(jnp.int32), (0, NP - N),
                 constant_values=NUM_GRAPHS)
    batch3 = bp.reshape(NRB, 1, R)
    batch3b = bp.reshape(NRB2, 1, RB2)
    batch_c = bp.reshape(1, NP)
    batch2 = bp.reshape(8, NP // 8)

    idx = _knn(xp, batch3, batch_c, batch2)
    idx_flat = idx.reshape(-1)

    h1pre = _mm(xp, Wg0)
    msg1 = _sc_gather(h1pre, idx_flat)
    h1, h2pre = _gcn_mm(msg1, h1pre, _row2d(bg0), Wg1)
    msg2 = _sc_gather(h2pre, idx_flat)
    t1, s1, ss1 = _f1(msg2, h2pre, _row2d(bg1), h1, pW1, _row2d(pb1))
    seg, cnt, s2, ss2 = _f2(t1, s1, ss1, _row2d(pg1), _row2d(pB1),
                            pW2, _row2d(pb2), batch3b)
    return _head(seg, cnt, s2, ss2, _row2d(pg2), _row2d(pB2),
                 mW1, _row2d(mb1), _row2d(mg1), _row2d(mB1),
                 mW2, _row2d(mb2), _row2d(mg2), _row2d(mB2),
                 mW3, _row2d(mb3))


# speedup vs baseline: 6.1274x; 1.0610x over previous
import functools
import jax
import jax.numpy as jnp
from jax import lax
from jax.experimental import pallas as pl
from jax.experimental.pallas import tpu as pltpu

try:
    from jax.experimental.pallas import tpu_sc as plsc
    _HAS_SC = True
except ImportError:
    _HAS_SC = False

N = 10000
IN = 128
HID = 256
OUT = 10
K = 16
NUM_GRAPHS = 64

NP = 10240
R = 256
C = 128
NRB = NP // R
RB2 = 512
NRB2 = NP // RB2
NORM = 1.0 / 17.0

BIGI = 2 ** 30


def _knn_body(xr_ref, xf_ref, br_ref, bc_ref, b2_ref, w_ref, out_ref,
              hp_ref, vbuf_ref, ni_ref):
    hp_ref[...] = jnp.dot(xr_ref[...], w_ref[...],
                          preferred_element_type=jnp.float32)
    br = br_ref[0, 0, :]
    b_lo = jnp.min(br)
    b_hi = jnp.max(br)
    b2 = b2_ref[...]
    lo_col = jnp.sum((b2 < b_lo).astype(jnp.int32))
    hi_col = jnp.sum((b2 <= b_hi).astype(jnp.int32))
    ch_lo = lo_col // C
    ch_hi = (hi_col + C - 1) // C

    def fill(t, carry):
        c0 = t * C
        xc = xf_ref[pl.ds(c0, C), :]
        bcc = bc_ref[0, pl.ds(c0, C)]
        dot = lax.dot_general(xr_ref[...], xc, (((1,), (1,)), ((), ())),
                              preferred_element_type=jnp.float32)
        sqc = jnp.sum(xc * xc, axis=1)
        s = sqc[None, :] - 2.0 * dot
        same = br[:, None] == bcc[None, :]
        vbuf_ref[:, pl.ds(c0, C)] = jnp.where(same, s, jnp.inf)
        return carry

    lax.fori_loop(ch_lo, ch_hi, fill, 0)

    def sweep(prev_mi, clear):
        def chunk(t, carry):
            m, mi = carry
            c0 = t * C
            v = vbuf_ref[:, pl.ds(c0, C)]
            ii = c0 + lax.broadcasted_iota(jnp.int32, (R, C), 1)
            if clear:
                v = jnp.where(ii == prev_mi, jnp.inf, v)
                vbuf_ref[:, pl.ds(c0, C)] = v
            mc = jnp.min(v, axis=1, keepdims=True)
            mic = jnp.min(jnp.where(v == mc, ii, BIGI), axis=1,
                          keepdims=True)
            mi = jnp.where(mc < m, mic, mi)
            m = jnp.minimum(m, mc)
            return m, mi

        return lax.fori_loop(ch_lo, ch_hi, chunk,
                             (jnp.full((R, 1), jnp.inf, jnp.float32),
                              jnp.zeros((R, 1), jnp.int32)))

    _, mi = sweep(None, False)
    ni_ref[:, 0:1] = mi
    for k in range(1, K):
        _, mi = sweep(mi, True)
        ni_ref[:, k:k + 1] = mi
    out_ref[...] = ni_ref[...]


def _knn(xp, batch3, batch_c, batch2, Wg0):
    return pl.pallas_call(
        _knn_body,
        grid=(NRB,),
        in_specs=[
            pl.BlockSpec((R, IN), lambda i: (i, 0)),
            pl.BlockSpec((NP, IN), lambda i: (0, 0)),
            pl.BlockSpec((1, 1, R), lambda i: (i, 0, 0)),
            pl.BlockSpec((1, NP), lambda i: (0, 0)),
            pl.BlockSpec((8, NP // 8), lambda i: (0, 0)),
            pl.BlockSpec((IN, HID), lambda i: (0, 0)),
        ],
        out_specs=[
            pl.BlockSpec((R, K), lambda i: (i, 0)),
            pl.BlockSpec((R, HID), lambda i: (i, 0)),
        ],
        out_shape=[
            jax.ShapeDtypeStruct((NP, K), jnp.int32),
            jax.ShapeDtypeStruct((NP, HID), jnp.float32),
        ],
        scratch_shapes=[
            pltpu.VMEM((R, NP), jnp.float32),
            pltpu.VMEM((R, K), jnp.int32),
        ],
        compiler_params=pltpu.CompilerParams(
            dimension_semantics=("arbitrary",)),
    )(xp, xp, batch3, batch_c, batch2, Wg0)


NW = 32
NPW = NP // NW
G = 8
NCHUNK = NPW // G


def _sc_gather_body(h_hbm, idx_hbm, out_hbm, idx0, idx1, rows0, rows1,
                    acc_v, sem0, sem1):
    c = lax.axis_index("c")
    s = lax.axis_index("s")
    wid = s * 2 + c
    base = wid * NPW

    def issue(t, idx_v, rows_v, sem):
        n0 = base + t * G
        pltpu.sync_copy(idx_hbm.at[pl.ds(n0 * K, G * K)], idx_v)
        pltpu.async_copy(h_hbm.at[idx_v], rows_v, sem)

    def drain_acc(t, idx_v, rows_v, sem):
        pltpu.make_async_copy(h_hbm.at[idx_v], rows_v, sem).wait()

        def col(ci, carry2):
            for g in range(G):
                acc = rows_v[g * K, pl.ds(ci * 16, 16)]
                for r in range(1, K):
                    acc = acc + rows_v[g * K + r, pl.ds(ci * 16, 16)]
                acc_v[g, pl.ds(ci * 16, 16)] = acc
            return carry2

        lax.fori_loop(0, HID // 16, col, 0)
        pltpu.sync_copy(acc_v, out_hbm.at[pl.ds(base + t * G, G)])

    issue(0, idx0, rows0, sem0)

    def pair(p, carry):
        t0 = p * 2
        issue(t0 + 1, idx1, rows1, sem1)
        drain_acc(t0, idx0, rows0, sem0)

        @pl.when(t0 + 2 < NCHUNK)
        def _():
            issue(t0 + 2, idx0, rows0, sem0)

        drain_acc(t0 + 1, idx1, rows1, sem1)
        return carry

    lax.fori_loop(0, NCHUNK // 2, pair, 0)


def _sc_gather(h, idx_flat):
    mesh = plsc.VectorSubcoreMesh(core_axis_name="c", subcore_axis_name="s")
    fn = functools.partial(
        pl.kernel,
        _sc_gather_body,
        mesh=mesh,
        out_type=jax.ShapeDtypeStruct((NP, HID), jnp.float32),
        scratch_types=[
            pltpu.VMEM((G * K,), jnp.int32),
            pltpu.VMEM((G * K,), jnp.int32),
            pltpu.VMEM((G * K, HID), jnp.float32),
            pltpu.VMEM((G * K, HID), jnp.float32),
            pltpu.VMEM((G, HID), jnp.float32),
            pltpu.SemaphoreType.DMA,
            pltpu.SemaphoreType.DMA,
        ],
    )()
    return fn(h, idx_flat)


def _gcn_mm_body(msg_ref, hpre_ref, b_ref, w_ref, h_out_ref, hp2_ref):
    z = (msg_ref[...] + hpre_ref[...]) * NORM + b_ref[...]
    h = jnp.where(z > 0, z, jnp.exp(jnp.where(z > 0, 0.0, z)) - 1.0)
    h_out_ref[...] = h
    hp2_ref[...] = jnp.dot(h, w_ref[...], preferred_element_type=jnp.float32)


def _gcn_mm(msg, hpre, b2d, W):
    return pl.pallas_call(
        _gcn_mm_body,
        grid=(NRB2,),
        in_specs=[
            pl.BlockSpec((RB2, HID), lambda i: (i, 0)),
            pl.BlockSpec((RB2, HID), lambda i: (i, 0)),
            pl.BlockSpec((1, HID), lambda i: (0, 0)),
            pl.BlockSpec((HID, HID), lambda i: (0, 0)),
        ],
        out_specs=[
            pl.BlockSpec((RB2, HID), lambda i: (i, 0)),
            pl.BlockSpec((RB2, HID), lambda i: (i, 0)),
        ],
        out_shape=[
            jax.ShapeDtypeStruct((NP, HID), jnp.float32),
            jax.ShapeDtypeStruct((NP, HID), jnp.float32),
        ],
    )(msg, hpre, b2d, W)


def _f1_body(msg_ref, hpre_ref, b_ref, h1_ref, w_ref, pb_ref,
             t1_ref, s_ref, ss_ref):
    i = pl.program_id(0)
    z = (msg_ref[...] + hpre_ref[...]) * NORM + b_ref[...]
    h2 = jnp.where(z > 0, z, jnp.exp(jnp.where(z > 0, 0.0, z)) - 1.0)
    hcat = jnp.concatenate([h1_ref[...], h2], axis=1)
    t1 = jnp.maximum(jnp.dot(hcat, w_ref[...],
                             preferred_element_type=jnp.float32)
                     + pb_ref[...], 0.0)
    rowid = i * RB2 + lax.broadcasted_iota(jnp.int32, (RB2, 1), 0)
    t1 = jnp.where(rowid < N, t1, 0.0)
    t1_ref[...] = t1

    @pl.when(i == 0)
    def _():
        s_ref[...] = jnp.zeros_like(s_ref)
        ss_ref[...] = jnp.zeros_like(ss_ref)

    s_ref[...] += jnp.sum(t1, axis=0, keepdims=True)
    ss_ref[...] += jnp.sum(t1 * t1, axis=0, keepdims=True)


def _f1(msg2, h2pre, b2d, h1, pW1, pb1):
    return pl.pallas_call(
        _f1_body,
        grid=(NRB2,),
        in_specs=[
            pl.BlockSpec((RB2, HID), lambda i: (i, 0)),
            pl.BlockSpec((RB2, HID), lambda i: (i, 0)),
            pl.BlockSpec((1, HID), lambda i: (0, 0)),
            pl.BlockSpec((RB2, HID), lambda i: (i, 0)),
            pl.BlockSpec((2 * HID, HID), lambda i: (0, 0)),
            pl.BlockSpec((1, HID), lambda i: (0, 0)),
        ],
        out_specs=[
            pl.BlockSpec((RB2, HID), lambda i: (i, 0)),
            pl.BlockSpec((1, HID), lambda i: (0, 0)),
            pl.BlockSpec((1, HID), lambda i: (0, 0)),
        ],
        out_shape=[
            jax.ShapeDtypeStruct((NP, HID), jnp.float32),
            jax.ShapeDtypeStruct((1, HID), jnp.float32),
            jax.ShapeDtypeStruct((1, HID), jnp.float32),
        ],
        compiler_params=pltpu.CompilerParams(
            dimension_semantics=("arbitrary",)),
    )(msg2, h2pre, b2d, h1, pW1, pb1)


def _f2_body(t1_ref, s1_ref, ss1_ref, g_ref, B_ref, w_ref, pb_ref, bt_ref,
             seg_ref, cnt_ref, s2_ref, ss2_ref):
    i = pl.program_id(0)
    mu1 = s1_ref[...] * (1.0 / N)
    var1 = ss1_ref[...] * (1.0 / N) - mu1 * mu1
    r1 = lax.rsqrt(var1 + 1e-5)
    t1n = (t1_ref[...] - mu1) * (r1 * g_ref[...]) + B_ref[...]
    t2 = jnp.maximum(jnp.dot(t1n, w_ref[...],
                             preferred_element_type=jnp.float32)
                     + pb_ref[...], 0.0)
    rowid = i * RB2 + lax.broadcasted_iota(jnp.int32, (RB2, 1), 0)
    t2 = jnp.where(rowid < N, t2, 0.0)
    br = bt_ref[0, 0, :]
    giota = lax.broadcasted_iota(jnp.int32, (RB2, NUM_GRAPHS), 1)
    oh = (br[:, None] == giota).astype(jnp.float32)

    @pl.when(i == 0)
    def _():
        seg_ref[...] = jnp.zeros_like(seg_ref)
        cnt_ref[...] = jnp.zeros_like(cnt_ref)
        s2_ref[...] = jnp.zeros_like(s2_ref)
        ss2_ref[...] = jnp.zeros_like(ss2_ref)

    seg_ref[...] += lax.dot_general(oh, t2, (((0,), (0,)), ((), ())),
                                    preferred_element_type=jnp.float32)
    cnt_ref[...] += lax.dot_general(
        oh, jnp.ones((RB2, 128), jnp.float32), (((0,), (0,)), ((), ())),
        preferred_element_type=jnp.float32)
    s2_ref[...] += jnp.sum(t2, axis=0, keepdims=True)
    ss2_ref[...] += jnp.sum(t2 * t2, axis=0, keepdims=True)


def _f2(t1, s1, ss1, pg1, pB1, pW2, pb2, batch3b):
    return pl.pallas_call(
        _f2_body,
        grid=(NRB2,),
        in_specs=[
            pl.BlockSpec((RB2, HID), lambda i: (i, 0)),
            pl.BlockSpec((1, HID), lambda i: (0, 0)),
            pl.BlockSpec((1, HID), lambda i: (0, 0)),
            pl.BlockSpec((1, HID), lambda i: (0, 0)),
            pl.BlockSpec((1, HID), lambda i: (0, 0)),
            pl.BlockSpec((HID, HID), lambda i: (0, 0)),
            pl.BlockSpec((1, HID), lambda i: (0, 0)),
            pl.BlockSpec((1, 1, RB2), lambda i: (i, 0, 0)),
        ],
        out_specs=[
            pl.BlockSpec((NUM_GRAPHS, HID), lambda i: (0, 0)),
            pl.BlockSpec((NUM_GRAPHS, 128), lambda i: (0, 0)),
            pl.BlockSpec((1, HID), lambda i: (0, 0)),
            pl.BlockSpec((1, HID), lambda i: (0, 0)),
        ],
        out_shape=[
            jax.ShapeDtypeStruct((NUM_GRAPHS, HID), jnp.float32),
            jax.ShapeDtypeStruct((NUM_GRAPHS, 128), jnp.float32),
            jax.ShapeDtypeStruct((1, HID), jnp.float32),
            jax.ShapeDtypeStruct((1, HID), jnp.float32),
        ],
        compiler_params=pltpu.CompilerParams(
            dimension_semantics=("arbitrary",)),
    )(t1, s1, ss1, pg1, pB1, pW2, pb2, batch3b)


def _head_body(seg_ref, cnt_ref, s2_ref, ss2_ref, g2_ref, B2_ref,
               w1_ref, b1_ref, g_ref, B_ref, w2_ref, b2_ref,
               gg_ref, BB_ref, w3_ref, b3_ref, o_ref):
    mu2 = s2_ref[...] * (1.0 / N)
    var2 = ss2_ref[...] * (1.0 / N) - mu2 * mu2
    r2 = lax.rsqrt(var2 + 1e-5)
    cnt = cnt_ref[:, 0:1]
    pooled = (seg_ref[...] - cnt * mu2) * (r2 * g2_ref[...]) \
        + cnt * B2_ref[...]

    def bn64(h, g, B):
        mu = jnp.mean(h, axis=0, keepdims=True)
        var = jnp.mean((h - mu) ** 2, axis=0, keepdims=True)
        return (h - mu) * lax.rsqrt(var + 1e-5) * g + B

    m = bn64(jnp.maximum(jnp.dot(pooled, w1_ref[...],
                                 preferred_element_type=jnp.float32)
                         + b1_ref[...], 0.0), g_ref[...], B_ref[...])
    m = bn64(jnp.maximum(jnp.dot(m, w2_ref[...],
                                 preferred_element_type=jnp.float32)
                         + b2_ref[...], 0.0), gg_ref[...], BB_ref[...])
    logits = jnp.dot(m, w3_ref[...],
                     preferred_element_type=jnp.float32) + b3_ref[...]
    lmax = jnp.max(logits, axis=1, keepdims=True)
    lz = logits - lmax
    o_ref[...] = lz - jnp.log(jnp.sum(jnp.exp(lz), axis=1, keepdims=True))


def _head(seg, cnt, s2, ss2, pg2, pB2, mW1, mb1, mg1, mB1,
          mW2, mb2, mg2, mB2, mW3, mb3):
    return pl.pallas_call(
        _head_body,
        out_shape=jax.ShapeDtypeStruct((NUM_GRAPHS, OUT), jnp.float32),
    )(seg, cnt, s2, ss2, pg2, pB2, mW1, mb1, mg1, mB1,
      mW2, mb2, mg2, mB2, mW3, mb3)


def _row2d(v):
    return v.reshape(1, -1)


def kernel(x, batch, Wg0, bg0, Wg1, bg1, pW1, pb1, pg1, pB1, pW2, pb2,
           pg2, pB2, mW1, mb1, mg1, mB1, mW2, mb2, mg2, mB2, mW3, mb3):
    xp = jnp.pad(x, ((0, NP - N), (0, 0)))
    bp = jnp.pad(batch.astype(jnp.int32), (0, NP - N),
                 constant_values=NUM_GRAPHS)
    batch3 = bp.reshape(NRB, 1, R)
    batch3b = bp.reshape(NRB2, 1, RB2)
    batch_c = bp.reshape(1, NP)
    batch2 = bp.reshape(8, NP // 8)

    idx, h1pre = _knn(xp, batch3, batch_c, batch2, Wg0)
    idx_flat = idx.reshape(-1)

    msg1 = _sc_gather(h1pre, idx_flat)
    h1, h2pre = _gcn_mm(msg1, h1pre, _row2d(bg0), Wg1)
    msg2 = _sc_gather(h2pre, idx_flat)
    t1, s1, ss1 = _f1(msg2, h2pre, _row2d(bg1), h1, pW1, _row2d(pb1))
    seg, cnt, s2, ss2 = _f2(t1, s1, ss1, _row2d(pg1), _row2d(pB1),
                            pW2, _row2d(pb2), batch3b)
    return _head(seg, cnt, s2, ss2, _row2d(pg2), _row2d(pB2),
                 mW1, _row2d(mb1), _row2d(mg1), _row2d(mB1),
                 mW2, _row2d(mb2), _row2d(mg2), _row2d(mB2),
                 mW3, _row2d(mb3))


# speedup vs baseline: 6.1485x; 1.0034x over previous
import functools
import jax
import jax.numpy as jnp
from jax import lax
from jax.experimental import pallas as pl
from jax.experimental.pallas import tpu as pltpu

try:
    from jax.experimental.pallas import tpu_sc as plsc
    _HAS_SC = True
except ImportError:
    _HAS_SC = False

N = 10000
IN = 128
HID = 256
OUT = 10
K = 16
NUM_GRAPHS = 64

NP = 10240
R = 256
C = 128
NRB = NP // R
RB2 = 512
NRB2 = NP // RB2
NORM = 1.0 / 17.0

BIGI = 2 ** 30


def _knn_body(xr_ref, xf_ref, br_ref, bc_ref, b2_ref, w_ref, out_ref,
              hp_ref, vbuf_ref, ni_ref):
    hp_ref[...] = jnp.dot(xr_ref[...], w_ref[...],
                          preferred_element_type=jnp.float32)
    br = br_ref[0, 0, :]
    b_lo = jnp.min(br)
    b_hi = jnp.max(br)
    b2 = b2_ref[...]
    lo_col = jnp.sum((b2 < b_lo).astype(jnp.int32))
    hi_col = jnp.sum((b2 <= b_hi).astype(jnp.int32))
    ch_lo = lo_col // C
    ch_hi = (hi_col + C - 1) // C

    def fill(t, carry):
        c0 = t * C
        xc = xf_ref[pl.ds(c0, C), :]
        bcc = bc_ref[0, pl.ds(c0, C)]
        dot = lax.dot_general(xr_ref[...], xc, (((1,), (1,)), ((), ())),
                              preferred_element_type=jnp.float32)
        sqc = jnp.sum(xc * xc, axis=1)
        s = sqc[None, :] - 2.0 * dot
        same = br[:, None] == bcc[None, :]
        vbuf_ref[:, pl.ds(c0, C)] = jnp.where(same, s, jnp.inf)
        return carry

    lax.fori_loop(ch_lo, ch_hi, fill, 0)

    def sweep(prev_mi, clear):
        def chunk(t, carry):
            m, mi = carry
            c0 = t * C
            v = vbuf_ref[:, pl.ds(c0, C)]
            ii = c0 + lax.broadcasted_iota(jnp.int32, (R, C), 1)
            if clear:
                v = jnp.where(ii == prev_mi, jnp.inf, v)
                vbuf_ref[:, pl.ds(c0, C)] = v
            mc = jnp.min(v, axis=1, keepdims=True)
            mic = jnp.min(jnp.where(v == mc, ii, BIGI), axis=1,
                          keepdims=True)
            mi = jnp.where(mc < m, mic, mi)
            m = jnp.minimum(m, mc)
            return m, mi

        return lax.fori_loop(ch_lo, ch_hi, chunk,
                             (jnp.full((R, 1), jnp.inf, jnp.float32),
                              jnp.zeros((R, 1), jnp.int32)))

    _, mi = sweep(None, False)
    ni_ref[:, 0:1] = mi
    for k in range(1, K):
        _, mi = sweep(mi, True)
        ni_ref[:, k:k + 1] = mi
    out_ref[...] = ni_ref[...]


def _knn(xp, batch3, batch_c, batch2, Wg0):
    return pl.pallas_call(
        _knn_body,
        grid=(NRB,),
        in_specs=[
            pl.BlockSpec((R, IN), lambda i: (i, 0)),
            pl.BlockSpec((NP, IN), lambda i: (0, 0)),
            pl.BlockSpec((1, 1, R), lambda i: (i, 0, 0)),
            pl.BlockSpec((1, NP), lambda i: (0, 0)),
            pl.BlockSpec((8, NP // 8), lambda i: (0, 0)),
            pl.BlockSpec((IN, HID), lambda i: (0, 0)),
        ],
        out_specs=[
            pl.BlockSpec((R, K), lambda i: (i, 0)),
            pl.BlockSpec((R, HID), lambda i: (i, 0)),
        ],
        out_shape=[
            jax.ShapeDtypeStruct((NP, K), jnp.int32),
            jax.ShapeDtypeStruct((NP, HID), jnp.float32),
        ],
        scratch_shapes=[
            pltpu.VMEM((R, NP), jnp.float32),
            pltpu.VMEM((R, K), jnp.int32),
        ],
        compiler_params=pltpu.CompilerParams(
            dimension_semantics=("arbitrary",)),
    )(xp, xp, batch3, batch_c, batch2, Wg0)


NW = 32
NPW = NP // NW
G = 8
NCHUNK = NPW // G


def _sc_gather_body(h_hbm, idx_hbm, out_hbm, idx0, idx1, rows0, rows1,
                    acc_v, sem0, sem1):
    c = lax.axis_index("c")
    s = lax.axis_index("s")
    wid = s * 2 + c
    base = wid * NPW

    def issue(t, idx_v, rows_v, sem):
        n0 = base + t * G
        pltpu.sync_copy(idx_hbm.at[pl.ds(n0 * K, G * K)], idx_v)
        pltpu.async_copy(h_hbm.at[idx_v], rows_v, sem)

    def drain_acc(t, idx_v, rows_v, sem):
        pltpu.make_async_copy(h_hbm.at[idx_v], rows_v, sem).wait()

        def col(ci, carry2):
            for g in range(G):
                acc = rows_v[g * K, pl.ds(ci * 16, 16)]
                for r in range(1, K):
                    acc = acc + rows_v[g * K + r, pl.ds(ci * 16, 16)]
                acc_v[g, pl.ds(ci * 16, 16)] = acc
            return carry2

        lax.fori_loop(0, HID // 16, col, 0)
        pltpu.sync_copy(acc_v, out_hbm.at[pl.ds(base + t * G, G)])

    issue(0, idx0, rows0, sem0)

    def pair(p, carry):
        t0 = p * 2
        issue(t0 + 1, idx1, rows1, sem1)
        drain_acc(t0, idx0, rows0, sem0)

        @pl.when(t0 + 2 < NCHUNK)
        def _():
            issue(t0 + 2, idx0, rows0, sem0)

        drain_acc(t0 + 1, idx1, rows1, sem1)
        return carry

    lax.fori_loop(0, NCHUNK // 2, pair, 0)


def _sc_gather(h, idx_flat):
    mesh = plsc.VectorSubcoreMesh(core_axis_name="c", subcore_axis_name="s")
    fn = functools.partial(
        pl.kernel,
        _sc_gather_body,
        mesh=mesh,
        out_type=jax.ShapeDtypeStruct((NP, HID), jnp.float32),
        scratch_types=[
            pltpu.VMEM((G * K,), jnp.int32),
            pltpu.VMEM((G * K,), jnp.int32),
            pltpu.VMEM((G * K, HID), jnp.float32),
            pltpu.VMEM((G * K, HID), jnp.float32),
            pltpu.VMEM((G, HID), jnp.float32),
            pltpu.SemaphoreType.DMA,
            pltpu.SemaphoreType.DMA,
        ],
    )()
    return fn(h, idx_flat)


def _gcn_mm_body(msg_ref, hpre_ref, b_ref, w_ref, h_out_ref, hp2_ref):
    z = (msg_ref[...] + hpre_ref[...]) * NORM + b_ref[...]
    h = jnp.where(z > 0, z, jnp.exp(jnp.where(z > 0, 0.0, z)) - 1.0)
    h_out_ref[...] = h
    hp2_ref[...] = jnp.dot(h, w_ref[...], preferred_element_type=jnp.float32)


def _gcn_mm(msg, hpre, b2d, W):
    return pl.pallas_call(
        _gcn_mm_body,
        grid=(NRB2,),
        in_specs=[
            pl.BlockSpec((RB2, HID), lambda i: (i, 0)),
            pl.BlockSpec((RB2, HID), lambda i: (i, 0)),
            pl.BlockSpec((1, HID), lambda i: (0, 0)),
            pl.BlockSpec((HID, HID), lambda i: (0, 0)),
        ],
        out_specs=[
            pl.BlockSpec((RB2, HID), lambda i: (i, 0)),
            pl.BlockSpec((RB2, HID), lambda i: (i, 0)),
        ],
        out_shape=[
            jax.ShapeDtypeStruct((NP, HID), jnp.float32),
            jax.ShapeDtypeStruct((NP, HID), jnp.float32),
        ],
    )(msg, hpre, b2d, W)


TAIL_STEPS = 2 * NRB2 + 1


def _tail_body(msg_ref, hpre_ref, b_ref, h1_ref, pw1_ref, pb1_ref,
               pg1_ref, pB1_ref, pw2_ref, pb2_ref, bt_ref,
               pg2_ref, pB2_ref, w1_ref, b1_ref, g1_ref, B1_ref,
               w2_ref, b2_ref, g2_ref, B2_ref, w3_ref, b3_ref,
               o_ref,
               t1_ref, s1_ref, ss1_ref, seg_ref, cnt_ref, s2_ref, ss2_ref):
    i = pl.program_id(0)

    @pl.when(i == 0)
    def _():
        s1_ref[...] = jnp.zeros_like(s1_ref)
        ss1_ref[...] = jnp.zeros_like(ss1_ref)
        seg_ref[...] = jnp.zeros_like(seg_ref)
        cnt_ref[...] = jnp.zeros_like(cnt_ref)
        s2_ref[...] = jnp.zeros_like(s2_ref)
        ss2_ref[...] = jnp.zeros_like(ss2_ref)

    @pl.when(i < NRB2)
    def _():
        z = (msg_ref[...] + hpre_ref[...]) * NORM + b_ref[...]
        h2 = jnp.where(z > 0, z, jnp.exp(jnp.where(z > 0, 0.0, z)) - 1.0)
        hcat = jnp.concatenate([h1_ref[...], h2], axis=1)
        t1 = jnp.maximum(jnp.dot(hcat, pw1_ref[...],
                                 preferred_element_type=jnp.float32)
                         + pb1_ref[...], 0.0)
        rowid = i * RB2 + lax.broadcasted_iota(jnp.int32, (RB2, 1), 0)
        t1 = jnp.where(rowid < N, t1, 0.0)
        t1_ref[pl.ds(i * RB2, RB2), :] = t1
        s1_ref[...] += jnp.sum(t1, axis=0, keepdims=True)
        ss1_ref[...] += jnp.sum(t1 * t1, axis=0, keepdims=True)

    @pl.when(jnp.logical_and(i >= NRB2, i < 2 * NRB2))
    def _():
        j = i - NRB2
        mu1 = s1_ref[...] * (1.0 / N)
        var1 = ss1_ref[...] * (1.0 / N) - mu1 * mu1
        r1 = lax.rsqrt(var1 + 1e-5)
        t1 = t1_ref[pl.ds(j * RB2, RB2), :]
        t1n = (t1 - mu1) * (r1 * pg1_ref[...]) + pB1_ref[...]
        t2 = jnp.maximum(jnp.dot(t1n, pw2_ref[...],
                                 preferred_element_type=jnp.float32)
                         + pb2_ref[...], 0.0)
        rowid = j * RB2 + lax.broadcasted_iota(jnp.int32, (RB2, 1), 0)
        t2 = jnp.where(rowid < N, t2, 0.0)
        br = bt_ref[0, 0, :]
        giota = lax.broadcasted_iota(jnp.int32, (RB2, NUM_GRAPHS), 1)
        oh = (br[:, None] == giota).astype(jnp.float32)
        seg_ref[...] += lax.dot_general(oh, t2, (((0,), (0,)), ((), ())),
                                        preferred_element_type=jnp.float32)
        cnt_ref[...] += lax.dot_general(
            oh, jnp.ones((RB2, 128), jnp.float32), (((0,), (0,)), ((), ())),
            preferred_element_type=jnp.float32)
        s2_ref[...] += jnp.sum(t2, axis=0, keepdims=True)
        ss2_ref[...] += jnp.sum(t2 * t2, axis=0, keepdims=True)

    @pl.when(i == 2 * NRB2)
    def _():
        mu2 = s2_ref[...] * (1.0 / N)
        var2 = ss2_ref[...] * (1.0 / N) - mu2 * mu2
        r2 = lax.rsqrt(var2 + 1e-5)
        cnt = cnt_ref[:, 0:1]
        pooled = (seg_ref[...] - cnt * mu2) * (r2 * pg2_ref[...]) \
            + cnt * pB2_ref[...]

        def bn64(h, g, B):
            mu = jnp.mean(h, axis=0, keepdims=True)
            var = jnp.mean((h - mu) ** 2, axis=0, keepdims=True)
            return (h - mu) * lax.rsqrt(var + 1e-5) * g + B

        m = bn64(jnp.maximum(jnp.dot(pooled, w1_ref[...],
                                     preferred_element_type=jnp.float32)
                             + b1_ref[...], 0.0), g1_ref[...], B1_ref[...])
        m = bn64(jnp.maximum(jnp.dot(m, w2_ref[...],
                                     preferred_element_type=jnp.float32)
                             + b2_ref[...], 0.0), g2_ref[...], B2_ref[...])
        logits = jnp.dot(m, w3_ref[...],
                         preferred_element_type=jnp.float32) + b3_ref[...]
        lmax = jnp.max(logits, axis=1, keepdims=True)
        lz = logits - lmax
        o_ref[...] = lz - jnp.log(jnp.sum(jnp.exp(lz), axis=1,
                                          keepdims=True))


def _tail(msg2, h2pre, b2d, h1, pW1, pb1, pg1, pB1, pW2, pb2, batch3b,
          pg2, pB2, mW1, mb1, mg1, mB1, mW2, mb2, mg2, mB2, mW3, mb3):
    blkA = lambda i: (jnp.where(i < NRB2, i, 0), 0)
    blkB3 = lambda i: (jnp.where(jnp.logical_and(i >= NRB2, i < 2 * NRB2),
                                 i - NRB2, 0), 0, 0)
    cst = lambda i: (0, 0)
    return pl.pallas_call(
        _tail_body,
        grid=(TAIL_STEPS,),
        in_specs=[
            pl.BlockSpec((RB2, HID), blkA),
            pl.BlockSpec((RB2, HID), blkA),
            pl.BlockSpec((1, HID), cst),
            pl.BlockSpec((RB2, HID), blkA),
            pl.BlockSpec((2 * HID, HID), cst),
            pl.BlockSpec((1, HID), cst),
            pl.BlockSpec((1, HID), cst),
            pl.BlockSpec((1, HID), cst),
            pl.BlockSpec((HID, HID), cst),
            pl.BlockSpec((1, HID), cst),
            pl.BlockSpec((1, 1, RB2), blkB3),
            pl.BlockSpec((1, HID), cst),
            pl.BlockSpec((1, HID), cst),
            pl.BlockSpec((HID, HID), cst),
            pl.BlockSpec((1, HID), cst),
            pl.BlockSpec((1, HID), cst),
            pl.BlockSpec((1, HID), cst),
            pl.BlockSpec((HID, HID), cst),
            pl.BlockSpec((1, HID), cst),
            pl.BlockSpec((1, HID), cst),
            pl.BlockSpec((1, HID), cst),
            pl.BlockSpec((HID, OUT), cst),
            pl.BlockSpec((1, OUT), cst),
        ],
        out_specs=pl.BlockSpec((NUM_GRAPHS, OUT), cst),
        out_shape=jax.ShapeDtypeStruct((NUM_GRAPHS, OUT), jnp.float32),
        scratch_shapes=[
            pltpu.VMEM((NP, HID), jnp.float32),
            pltpu.VMEM((1, HID), jnp.float32),
            pltpu.VMEM((1, HID), jnp.float32),
            pltpu.VMEM((NUM_GRAPHS, HID), jnp.float32),
            pltpu.VMEM((NUM_GRAPHS, 128), jnp.float32),
            pltpu.VMEM((1, HID), jnp.float32),
            pltpu.VMEM((1, HID), jnp.float32),
        ],
        compiler_params=pltpu.CompilerParams(
            dimension_semantics=("arbitrary",)),
    )(msg2, h2pre, b2d, h1, pW1, pb1, pg1, pB1, pW2, pb2, batch3b,
      pg2, pB2, mW1, mb1, mg1, mB1, mW2, mb2, mg2, mB2, mW3, mb3)


def _row2d(v):
    return v.reshape(1, -1)


def kernel(x, batch, Wg0, bg0, Wg1, bg1, pW1, pb1, pg1, pB1, pW2, pb2,
           pg2, pB2, mW1, mb1, mg1, mB1, mW2, mb2, mg2, mB2, mW3, mb3):
    xp = jnp.pad(x, ((0, NP - N), (0, 0)))
    bp = jnp.pad(batch.astype(jnp.int32), (0, NP - N),
                 constant_values=NUM_GRAPHS)
    batch3 = bp.reshape(NRB, 1, R)
    batch3b = bp.reshape(NRB2, 1, RB2)
    batch_c = bp.reshape(1, NP)
    batch2 = bp.reshape(8, NP // 8)

    idx, h1pre = _knn(xp, batch3, batch_c, batch2, Wg0)
    idx_flat = idx.reshape(-1)

    msg1 = _sc_gather(h1pre, idx_flat)
    h1, h2pre = _gcn_mm(msg1, h1pre, _row2d(bg0), Wg1)
    msg2 = _sc_gather(h2pre, idx_flat)
    return _tail(msg2, h2pre, _row2d(bg1), h1, pW1, _row2d(pb1),
                 _row2d(pg1), _row2d(pB1), pW2, _row2d(pb2), batch3b,
                 _row2d(pg2), _row2d(pB2),
                 mW1, _row2d(mb1), _row2d(mg1), _row2d(mB1),
                 mW2, _row2d(mb2), _row2d(mg2), _row2d(mB2),
                 mW3, _row2d(mb3))
